# baseline (speedup 1.0000x reference)
import jax
import jax.numpy as jnp
from jax.experimental import pallas as pl
from jax.experimental.pallas import tpu as pltpu

_B = 32
_NCLS = 91
_R = 1000
_CC = 7
_CB = 13


def _body(cls_ref, tb_ref, pred_ref, out_ref, psel, acc):
    b = pl.program_id(0)
    cc = pl.program_id(1)

    @pl.when(jnp.logical_and(b == 0, cc == 0))
    def _init():
        acc[0] = 0.0
        acc[1] = 0.0

    cls_row = cls_ref[0]

    part = jnp.zeros((4, _R), jnp.float32)
    for j in range(_CB):
        cid = cc * _CB + j
        part = part + jnp.where(cls_row == cid, pred_ref[0, j], 0.0)

    @pl.when(cc == 0)
    def _first():
        psel[...] = part

    @pl.when(cc > 0)
    def _rest():
        psel[...] = psel[...] + part

    @pl.when(cc == _CC - 1)
    def _loss():
        tb = tb_ref[0]
        d = jnp.abs(tb - psel[...])
        l = jnp.where(d < 1.0, 0.5 * d * d, d - 0.5)
        valid = cls_row > 0
        acc[0] = acc[0] + jnp.sum(jnp.where(valid, l, 0.0))
        acc[1] = acc[1] + 4.0 * jnp.sum(valid.astype(jnp.float32))

    @pl.when(jnp.logical_and(b == _B - 1, cc == _CC - 1))
    def _fin():
        total, count = acc[0], acc[1]
        out_ref[...] = jnp.reshape(
            jnp.where(count > 0, total / jnp.maximum(count, 1.0), 0.0), (1, 1))


def kernel(target_boxes, target_class_ids, pred_boxes):
    cls = target_class_ids.astype(jnp.int32).reshape(_B, 1, _R)
    tb = target_boxes.transpose(0, 2, 1)
    pred = pred_boxes.transpose(0, 2, 3, 1)

    out = pl.pallas_call(
        _body,
        grid=(_B, _CC),
        in_specs=[
            pl.BlockSpec((1, 1, _R), lambda b, cc: (b, 0, 0)),
            pl.BlockSpec((1, 4, _R), lambda b, cc: (b, 0, 0)),
            pl.BlockSpec((1, _CB, 4, _R), lambda b, cc: (b, cc, 0, 0)),
        ],
        out_specs=pl.BlockSpec((1, 1), lambda b, cc: (0, 0)),
        out_shape=jax.ShapeDtypeStruct((1, 1), jnp.float32),
        scratch_shapes=[pltpu.VMEM((4, _R), jnp.float32),
                        pltpu.SMEM((2,), jnp.float32)],
    )(cls, tb, pred)
    return out[0, 0]


# speedup vs baseline: 4.4769x; 4.4769x over previous
import jax
import jax.numpy as jnp
from jax.experimental import pallas as pl
from jax.experimental.pallas import tpu as pltpu

_B = 32
_NCLS = 91
_R = 1000


def _body(cls_ref, tb_ref, pred_ref, out_ref, acc):
    b = pl.program_id(0)

    @pl.when(b == 0)
    def _init():
        acc[0] = 0.0
        acc[1] = 0.0

    for i in range(2):
        cls_row = cls_ref[i]
        tb = tb_ref[i]
        psel = jnp.zeros((4, _R), jnp.float32)
        for c in range(1, _NCLS):
            psel = psel + jnp.where(cls_row == c, pred_ref[i, c], 0.0)
        d = jnp.abs(tb - psel)
        l = jnp.where(d < 1.0, 0.5 * d * d, d - 0.5)
        valid = cls_row > 0
        acc[0] = acc[0] + jnp.sum(jnp.where(valid, l, 0.0))
        acc[1] = acc[1] + 4.0 * jnp.sum(valid.astype(jnp.float32))

    @pl.when(b == _B // 2 - 1)
    def _fin():
        total, count = acc[0], acc[1]
        out_ref[...] = jnp.reshape(
            jnp.where(count > 0, total / jnp.maximum(count, 1.0), 0.0), (1, 1))


def kernel(target_boxes, target_class_ids, pred_boxes):
    cls = target_class_ids.astype(jnp.int32).reshape(_B, 1, _R)
    tb = target_boxes.transpose(0, 2, 1)
    pred = pred_boxes.transpose(0, 2, 3, 1)

    out = pl.pallas_call(
        _body,
        grid=(_B // 2,),
        in_specs=[
            pl.BlockSpec((2, 1, _R), lambda b: (b, 0, 0)),
            pl.BlockSpec((2, 4, _R), lambda b: (b, 0, 0)),
            pl.BlockSpec((2, _NCLS, 4, _R), lambda b: (b, 0, 0, 0)),
        ],
        out_specs=pl.BlockSpec((1, 1), lambda b: (0, 0)),
        out_shape=jax.ShapeDtypeStruct((1, 1), jnp.float32),
        scratch_shapes=[pltpu.SMEM((2,), jnp.float32)],
    )(cls, tb, pred)
    return out[0, 0]


# speedup vs baseline: 5.0393x; 1.1256x over previous
import jax
import jax.numpy as jnp
from jax.experimental import pallas as pl
from jax.experimental.pallas import tpu as pltpu

_B = 32
_NCLS = 91
_R = 1000


def _body(cls_ref, tb_ref, pred_ref, out_ref, acc):
    b = pl.program_id(0)

    @pl.when(b == 0)
    def _init():
        acc[0] = 0.0
        acc[1] = 0.0

    for i in range(4):
        cls_row = cls_ref[i]
        tb = tb_ref[i]
        psel = jnp.zeros((4, _R), jnp.float32)
        for c in range(1, _NCLS):
            psel = psel + jnp.where(cls_row == c, pred_ref[i, c], 0.0)
        d = jnp.abs(tb - psel)
        l = jnp.where(d < 1.0, 0.5 * d * d, d - 0.5)
        valid = cls_row > 0
        acc[0] = acc[0] + jnp.sum(jnp.where(valid, l, 0.0))
        acc[1] = acc[1] + 4.0 * jnp.sum(valid.astype(jnp.float32))

    @pl.when(b == _B // 4 - 1)
    def _fin():
        total, count = acc[0], acc[1]
        out_ref[...] = jnp.reshape(
            jnp.where(count > 0, total / jnp.maximum(count, 1.0), 0.0), (1, 1))


def kernel(target_boxes, target_class_ids, pred_boxes):
    cls = target_class_ids.astype(jnp.int32).reshape(_B, 1, _R)
    tb = target_boxes.transpose(0, 2, 1)
    pred = pred_boxes.transpose(0, 2, 3, 1)

    out = pl.pallas_call(
        _body,
        grid=(_B // 4,),
        in_specs=[
            pl.BlockSpec((4, 1, _R), lambda b: (b, 0, 0)),
            pl.BlockSpec((4, 4, _R), lambda b: (b, 0, 0)),
            pl.BlockSpec((4, _NCLS, 4, _R), lambda b: (b, 0, 0, 0)),
        ],
        out_specs=pl.BlockSpec((1, 1), lambda b: (0, 0)),
        out_shape=jax.ShapeDtypeStruct((1, 1), jnp.float32),
        scratch_shapes=[pltpu.SMEM((2,), jnp.float32)],
    )(cls, tb, pred)
    return out[0, 0]


# speedup vs baseline: 5.0533x; 1.0028x over previous
import jax
import jax.numpy as jnp
from jax.experimental import pallas as pl
from jax.experimental.pallas import tpu as pltpu

_B = 32
_NCLS = 91
_R = 1000


def _body(cls_ref, tb_ref, pred_ref, out_ref, acc):
    b = pl.program_id(0)

    @pl.when(b == 0)
    def _init():
        acc[0] = 0.0
        acc[1] = 0.0

    for i in range(8):
        cls_row = cls_ref[i]
        tb = tb_ref[i]
        psel = jnp.zeros((4, _R), jnp.float32)
        for c in range(1, _NCLS):
            psel = psel + jnp.where(cls_row == c, pred_ref[i, c], 0.0)
        d = jnp.abs(tb - psel)
        l = jnp.where(d < 1.0, 0.5 * d * d, d - 0.5)
        valid = cls_row > 0
        acc[0] = acc[0] + jnp.sum(jnp.where(valid, l, 0.0))
        acc[1] = acc[1] + 4.0 * jnp.sum(valid.astype(jnp.float32))

    @pl.when(b == _B // 8 - 1)
    def _fin():
        total, count = acc[0], acc[1]
        out_ref[...] = jnp.reshape(
            jnp.where(count > 0, total / jnp.maximum(count, 1.0), 0.0), (1, 1))


def kernel(target_boxes, target_class_ids, pred_boxes):
    cls = target_class_ids.astype(jnp.int32).reshape(_B, 1, _R)
    tb = target_boxes.transpose(0, 2, 1)
    pred = pred_boxes.transpose(0, 2, 3, 1)

    out = pl.pallas_call(
        _body,
        grid=(_B // 8,),
        in_specs=[
            pl.BlockSpec((8, 1, _R), lambda b: (b, 0, 0)),
            pl.BlockSpec((8, 4, _R), lambda b: (b, 0, 0)),
            pl.BlockSpec((8, _NCLS, 4, _R), lambda b: (b, 0, 0, 0)),
        ],
        out_specs=pl.BlockSpec((1, 1), lambda b: (0, 0)),
        out_shape=jax.ShapeDtypeStruct((1, 1), jnp.float32),
        scratch_shapes=[pltpu.SMEM((2,), jnp.float32)],
    )(cls, tb, pred)
    return out[0, 0]
